# baseline (speedup 1.0000x reference)
import functools
import math

import jax
import jax.numpy as jnp
from jax import lax
from jax.experimental import pallas as pl
from jax.experimental.pallas import tpu as pltpu
from jax.experimental.pallas import tpu_sc as plsc

_N = 10000
_E = 160000
_D = 256
_DA = 288
_NPAD = 10240
_NPS = 640
_NW = 32
_EPW = _E // _NW
_K = 125
_CH = _EPW // _K
_NV = _DA // 32

_mesh = plsc.VectorSubcoreMesh(core_axis_name="c", subcore_axis_name="s")


@functools.partial(
    pl.kernel,
    out_type=jax.ShapeDtypeStruct((2, _NPAD), jnp.float32),
    mesh=_mesh,
    scratch_types=[
        pltpu.VMEM((_CH, _K), jnp.int32),
        pltpu.VMEM((128,), jnp.float32),
        pltpu.VMEM((_NPS,), jnp.float32),
        pltpu.VMEM_SHARED((_NPAD,), jnp.float32),
        pltpu.SemaphoreType.DMA,
    ],
)
def _deg_kernel(row_hbm, deg_out, idx_v, ones_v, z_v, deg_sh, sem):
    c = lax.axis_index("c")
    s = lax.axis_index("s")
    wid = c * 16 + s
    zero16 = jnp.zeros((16,), jnp.float32)
    one16 = jnp.ones((16,), jnp.float32)
    for j in range(_NPS // 16):
        z_v[pl.ds(j * 16, 16)] = zero16
    for j in range(8):
        ones_v[pl.ds(j * 16, 16)] = one16
    pltpu.sync_copy(z_v, deg_sh.at[pl.ds(s * _NPS, _NPS)])
    plsc.subcore_barrier()
    pltpu.sync_copy(row_hbm.at[wid], idx_v)
    src = ones_v.at[pl.ds(0, _K)]
    for j in range(_CH):
        pltpu.async_copy(src, deg_sh.at[idx_v.at[j]], sem, add=True)
    for j in range(_CH):
        pltpu.make_async_copy(src, deg_sh.at[idx_v.at[j]], sem).wait()
    plsc.subcore_barrier()
    pltpu.sync_copy(deg_sh.at[pl.ds(s * _NPS, _NPS)],
                    deg_out.at[c, pl.ds(s * _NPS, _NPS)])


def _prep_body(logits_ref, labels_ref, x_ref, bl_ref, d0_ref, d1_ref,
               ta_ref, tb_ref, sc_ref):
    logits = logits_ref[...]
    g, cdim = logits.shape
    m = jnp.max(logits, axis=1, keepdims=True)
    lse = m + jnp.log(jnp.sum(jnp.exp(logits - m), axis=1, keepdims=True))
    onehot = lax.broadcasted_iota(jnp.int32, (g, cdim), 1) == labels_ref[...]
    picked = jnp.sum(jnp.where(onehot, logits, 0.0), axis=1, keepdims=True)
    ce = jnp.sum(lse - picked) / g
    inv_ng = 1.0 / (bl_ref[0, 0] + 1).astype(jnp.float32)
    sc_ref[...] = jnp.concatenate(
        [ce.reshape(1, 1), inv_ng.reshape(1, 1)], axis=1)

    x = x_ref[...]
    deg = d0_ref[: _N] + d1_ref[: _N]
    d = jnp.where(deg > 0, lax.rsqrt(jnp.maximum(deg, 1e-30)), 0.0)
    s = jnp.sum(x * x, axis=1, keepdims=True)
    r2 = math.sqrt(2.0)
    pad = jnp.zeros((_N, _DA - _D - 2), jnp.float32)
    ta_ref[...] = jnp.concatenate(
        [(-r2) * d * x, d * s, d, pad], axis=1).astype(jnp.bfloat16)
    tb_ref[...] = jnp.concatenate(
        [r2 * d * x, d, d * s, pad], axis=1).astype(jnp.bfloat16)


def _prep_call(logits, labels2, x, bl, deg0, deg1):
    return pl.pallas_call(
        _prep_body,
        out_shape=[
            jax.ShapeDtypeStruct((_N, _DA), jnp.bfloat16),
            jax.ShapeDtypeStruct((_N, _DA), jnp.bfloat16),
            jax.ShapeDtypeStruct((1, 2), jnp.float32),
        ],
        compiler_params=pltpu.CompilerParams(
            vmem_limit_bytes=100 * 1024 * 1024),
    )(logits, labels2, x, bl, deg0, deg1)


@functools.partial(
    pl.kernel,
    out_type=jax.ShapeDtypeStruct((_NW, 16), jnp.float32),
    mesh=_mesh,
    scratch_types=[
        pltpu.VMEM((_CH, _K), jnp.int32),
        pltpu.VMEM((_CH, _K), jnp.int32),
        pltpu.VMEM((_K, _DA // 2), jnp.int32),
        pltpu.VMEM((_K, _DA // 2), jnp.int32),
        pltpu.VMEM((_K, _DA // 2), jnp.int32),
        pltpu.VMEM((_K, _DA // 2), jnp.int32),
        pltpu.VMEM((16,), jnp.float32),
        pltpu.SemaphoreType.DMA,
        pltpu.SemaphoreType.DMA,
        pltpu.SemaphoreType.DMA,
        pltpu.SemaphoreType.DMA,
    ],
    compiler_params=pltpu.CompilerParams(
        use_tc_tiling_on_sc=False, needs_layout_passes=False),
)
def _energy_kernel(ta_hbm, tb_hbm, row_hbm, col_hbm, out,
                   idxr, idxc, a0, a1, b0, b1, res_v,
                   sa0, sa1, sb0, sb1):
    c = lax.axis_index("c")
    s = lax.axis_index("s")
    wid = c * 16 + s
    pltpu.sync_copy(row_hbm.at[wid], idxr)
    pltpu.sync_copy(col_hbm.at[wid], idxc)
    bufa = (a0, a1)
    bufb = (b0, b1)
    sema = (sa0, sa1)
    semb = (sb0, sb1)

    def start(chunk, b):
        pltpu.async_copy(ta_hbm.at[idxr.at[chunk]], bufa[b], sema[b])
        pltpu.async_copy(tb_hbm.at[idxc.at[chunk]], bufb[b], semb[b])

    def wait(chunk, b):
        pltpu.make_async_copy(ta_hbm.at[idxr.at[chunk]], bufa[b], sema[b]).wait()
        pltpu.make_async_copy(tb_hbm.at[idxc.at[chunk]], bufb[b], semb[b]).wait()

    start(0, 0)
    start(1, 1)
    zero = jnp.zeros((16,), jnp.float32)
    res_v[...] = zero

    def compute(b):
        def edge_body(e, accs):
            accs = list(accs)
            for v in range(_NV):
                av = plsc.bitcast(bufa[b][e, pl.ds(v * 16, 16)], jnp.bfloat16)
                bv = plsc.bitcast(bufb[b][e, pl.ds(v * 16, 16)], jnp.bfloat16)
                t0, t1 = plsc.unpack(av * bv, format=plsc.PackFormat.INTERLEAVED)
                accs[v % 4] = accs[v % 4] + (t0 + t1)
            return tuple(accs)

        accs = lax.fori_loop(0, _K, edge_body, (zero, zero, zero, zero),
                             unroll=2)
        res_v[...] = res_v[...] + accs[0] + accs[1] + accs[2] + accs[3]

    def outer(g2, carry):
        for b in range(2):
            chunk = g2 * 2 + b
            wait(chunk, b)
            compute(b)

            @pl.when(chunk + 2 < _CH)
            def _():
                start(chunk + 2, b)
        return carry

    lax.fori_loop(0, _CH // 2, outer, 0)
    pltpu.sync_copy(res_v, out.at[wid])


def kernel(logits, labels, x, edge_index, batch):
    labels2 = labels.astype(jnp.int32).reshape(-1, 1)
    row3 = edge_index[0].reshape(_NW, _CH, _K)
    col3 = edge_index[1].reshape(_NW, _CH, _K)
    bl = batch[-1:].astype(jnp.int32).reshape(1, 1)

    deg2 = _deg_kernel(row3)
    ta, tb, scal = _prep_call(
        logits, labels2, x, bl,
        deg2[0].reshape(_NPAD, 1), deg2[1].reshape(_NPAD, 1))
    taw = lax.bitcast_convert_type(
        ta.reshape(_N, _DA // 2, 2), jnp.int32)
    tbw = lax.bitcast_convert_type(
        tb.reshape(_N, _DA // 2, 2), jnp.int32)
    partials = _energy_kernel(taw, tbw, row3, col3)
    return scal[0, 0] + scal[0, 1] * jnp.sum(partials)


# speedup vs baseline: 2.0875x; 2.0875x over previous
import functools
import math

import jax
import jax.numpy as jnp
from jax import lax
from jax.experimental import pallas as pl
from jax.experimental.pallas import tpu as pltpu
from jax.experimental.pallas import tpu_sc as plsc

_N = 10000
_E = 160000
_D = 256
_DA = 288
_NPAD = 10240
_NPS = 640
_NW = 32
_EPW = _E // _NW
_K = 125
_CH = _EPW // _K
_NV = _DA // 32

_mesh = plsc.VectorSubcoreMesh(core_axis_name="c", subcore_axis_name="s")


@functools.partial(
    pl.kernel,
    out_type=jax.ShapeDtypeStruct((2, _NPAD), jnp.float32),
    mesh=_mesh,
    scratch_types=[
        pltpu.VMEM((_CH, _K), jnp.int32),
        pltpu.VMEM((128,), jnp.float32),
        pltpu.VMEM((_NPS,), jnp.float32),
        pltpu.VMEM_SHARED((_NPAD,), jnp.float32),
        pltpu.SemaphoreType.DMA,
    ],
)
def _deg_kernel(row_hbm, deg_out, idx_v, ones_v, z_v, deg_sh, sem):
    c = lax.axis_index("c")
    s = lax.axis_index("s")
    wid = c * 16 + s
    zero16 = jnp.zeros((16,), jnp.float32)
    one16 = jnp.ones((16,), jnp.float32)
    for j in range(_NPS // 16):
        z_v[pl.ds(j * 16, 16)] = zero16
    for j in range(8):
        ones_v[pl.ds(j * 16, 16)] = one16
    pltpu.sync_copy(z_v, deg_sh.at[pl.ds(s * _NPS, _NPS)])
    plsc.subcore_barrier()
    pltpu.sync_copy(row_hbm.at[wid], idx_v)
    src = ones_v.at[pl.ds(0, _K)]
    for j in range(_CH):
        pltpu.async_copy(src, deg_sh.at[idx_v.at[j]], sem, add=True)
    for j in range(_CH):
        pltpu.make_async_copy(src, deg_sh.at[idx_v.at[j]], sem).wait()
    plsc.subcore_barrier()
    pltpu.sync_copy(deg_sh.at[pl.ds(s * _NPS, _NPS)],
                    deg_out.at[c, pl.ds(s * _NPS, _NPS)])


def _prep_body(logits_ref, labels_ref, x_ref, bl_ref, d0_ref, d1_ref,
               ta_ref, tb_ref, sc_ref):
    logits = logits_ref[...]
    g, cdim = logits.shape
    m = jnp.max(logits, axis=1, keepdims=True)
    lse = m + jnp.log(jnp.sum(jnp.exp(logits - m), axis=1, keepdims=True))
    onehot = lax.broadcasted_iota(jnp.int32, (g, cdim), 1) == labels_ref[...]
    picked = jnp.sum(jnp.where(onehot, logits, 0.0), axis=1, keepdims=True)
    ce = jnp.sum(lse - picked) / g
    inv_ng = 1.0 / (bl_ref[0, 0] + 1).astype(jnp.float32)
    sc_ref[...] = jnp.concatenate(
        [ce.reshape(1, 1), inv_ng.reshape(1, 1)], axis=1)

    x = x_ref[...]
    bn = x.shape[0]
    deg = d0_ref[...] + d1_ref[...]
    d = jnp.where(deg > 0, lax.rsqrt(jnp.maximum(deg, 1e-30)), 0.0)
    s = jnp.sum(x * x, axis=1, keepdims=True)
    r2 = math.sqrt(2.0)
    pad = jnp.zeros((bn, _DA - _D - 2), jnp.float32)

    def pack_words(full):
        h = _DA // 2
        lo = lax.bitcast_convert_type(
            full[:, :h].astype(jnp.bfloat16), jnp.uint16).astype(jnp.uint32)
        hi = lax.bitcast_convert_type(
            full[:, h:].astype(jnp.bfloat16), jnp.uint16).astype(jnp.uint32)
        return lax.bitcast_convert_type((hi << 16) | lo, jnp.int32)

    ta_ref[...] = pack_words(
        jnp.concatenate([(-r2) * d * x, d * s, d, pad], axis=1))
    tb_ref[...] = pack_words(
        jnp.concatenate([r2 * d * x, d, d * s, pad], axis=1))


def _prep_call(logits, labels2, x, bl, deg0, deg1):
    bn = 2000
    nb = _N // bn
    return pl.pallas_call(
        _prep_body,
        grid=(nb,),
        in_specs=[
            pl.BlockSpec(logits.shape, lambda i: (0, 0)),
            pl.BlockSpec(labels2.shape, lambda i: (0, 0)),
            pl.BlockSpec((bn, _D), lambda i: (i, 0)),
            pl.BlockSpec((1, 1), lambda i: (0, 0)),
            pl.BlockSpec((bn, 1), lambda i: (i, 0)),
            pl.BlockSpec((bn, 1), lambda i: (i, 0)),
        ],
        out_specs=[
            pl.BlockSpec((bn, _DA // 2), lambda i: (i, 0)),
            pl.BlockSpec((bn, _DA // 2), lambda i: (i, 0)),
            pl.BlockSpec((1, 2), lambda i: (0, 0)),
        ],
        out_shape=[
            jax.ShapeDtypeStruct((_N, _DA // 2), jnp.int32),
            jax.ShapeDtypeStruct((_N, _DA // 2), jnp.int32),
            jax.ShapeDtypeStruct((1, 2), jnp.float32),
        ],
        compiler_params=pltpu.CompilerParams(
            vmem_limit_bytes=100 * 1024 * 1024),
    )(logits, labels2, x, bl, deg0, deg1)


@functools.partial(
    pl.kernel,
    out_type=jax.ShapeDtypeStruct((_NW, 16), jnp.float32),
    mesh=_mesh,
    scratch_types=[
        pltpu.VMEM((_CH, _K), jnp.int32),
        pltpu.VMEM((_CH, _K), jnp.int32),
        pltpu.VMEM((_K, _DA // 2), jnp.int32),
        pltpu.VMEM((_K, _DA // 2), jnp.int32),
        pltpu.VMEM((_K, _DA // 2), jnp.int32),
        pltpu.VMEM((_K, _DA // 2), jnp.int32),
        pltpu.VMEM((16,), jnp.float32),
        pltpu.SemaphoreType.DMA,
        pltpu.SemaphoreType.DMA,
        pltpu.SemaphoreType.DMA,
        pltpu.SemaphoreType.DMA,
    ],
    compiler_params=pltpu.CompilerParams(
        use_tc_tiling_on_sc=False, needs_layout_passes=False),
)
def _energy_kernel(ta_hbm, tb_hbm, row_hbm, col_hbm, out,
                   idxr, idxc, a0, a1, b0, b1, res_v,
                   sa0, sa1, sb0, sb1):
    c = lax.axis_index("c")
    s = lax.axis_index("s")
    wid = c * 16 + s
    pltpu.sync_copy(row_hbm.at[wid], idxr)
    pltpu.sync_copy(col_hbm.at[wid], idxc)
    bufa = (a0, a1)
    bufb = (b0, b1)
    sema = (sa0, sa1)
    semb = (sb0, sb1)

    def start(chunk, b):
        pltpu.async_copy(ta_hbm.at[idxr.at[chunk]], bufa[b], sema[b])
        pltpu.async_copy(tb_hbm.at[idxc.at[chunk]], bufb[b], semb[b])

    def wait(chunk, b):
        pltpu.make_async_copy(ta_hbm.at[idxr.at[chunk]], bufa[b], sema[b]).wait()
        pltpu.make_async_copy(tb_hbm.at[idxc.at[chunk]], bufb[b], semb[b]).wait()

    start(0, 0)
    start(1, 1)
    zero = jnp.zeros((16,), jnp.float32)
    res_v[...] = zero

    def compute(b):
        def edge_body(e, accs):
            accs = list(accs)
            for v in range(_NV):
                av = plsc.bitcast(bufa[b][e, pl.ds(v * 16, 16)], jnp.bfloat16)
                bv = plsc.bitcast(bufb[b][e, pl.ds(v * 16, 16)], jnp.bfloat16)
                t0, t1 = plsc.unpack(av * bv, format=plsc.PackFormat.INTERLEAVED)
                accs[v % 4] = accs[v % 4] + (t0 + t1)
            return tuple(accs)

        accs = lax.fori_loop(0, _K, edge_body, (zero, zero, zero, zero),
                             unroll=2)
        res_v[...] = res_v[...] + accs[0] + accs[1] + accs[2] + accs[3]

    def outer(g2, carry):
        for b in range(2):
            chunk = g2 * 2 + b
            wait(chunk, b)
            compute(b)

            @pl.when(chunk + 2 < _CH)
            def _():
                start(chunk + 2, b)
        return carry

    lax.fori_loop(0, _CH // 2, outer, 0)
    pltpu.sync_copy(res_v, out.at[wid])


def kernel(logits, labels, x, edge_index, batch):
    labels2 = labels.astype(jnp.int32).reshape(-1, 1)
    row3 = edge_index[0].reshape(_NW, _CH, _K)
    col3 = edge_index[1].reshape(_NW, _CH, _K)
    bl = batch[-1:].astype(jnp.int32).reshape(1, 1)

    deg2 = _deg_kernel(row3)
    taw, tbw, scal = _prep_call(
        logits, labels2, x, bl,
        deg2[0].reshape(_NPAD, 1), deg2[1].reshape(_NPAD, 1))
    partials = _energy_kernel(taw, tbw, row3, col3)
    return scal[0, 0] + scal[0, 1] * jnp.sum(partials)
